# Initial kernel scaffold; baseline (speedup 1.0000x reference)
import jax
import jax.numpy as jnp
from jax.experimental import pallas as pl


def kernel(node_type, c, gm, pos, r, vid, edge_index, batch, W1, b1, W2, b2, gcn_params, Wout, bout):
    raise NotImplementedError("write your pallas kernel here")


# speedup vs baseline: 9.8314x; 9.8314x over previous
import functools

import jax
import jax.numpy as jnp
from jax import lax
from jax.experimental import pallas as pl
from jax.experimental.pallas import tpu as pltpu
from jax.experimental.pallas import tpu_sc as plsc

N = 50000
E = 800000
H = 64
T = 16
NG = 128
NL = 5
Q = 16

RPT_ALL = 200
EPR = 32 * RPT_ALL
EP = EPR * 128
RPT_SC = EPR // 16
NP_ = 50176
RPN = NP_ // 16
ZR = 112
RB = 512
GRID = NP_ // RB
NBUF = 4
CH = 80
NCH = RPT_SC // CH

_mesh = plsc.VectorSubcoreMesh(core_axis_name="c", subcore_axis_name="s")


def _zeros16():
    return jnp.zeros((16,), jnp.float32)


@functools.partial(
    pl.kernel,
    mesh=_mesh,
    out_type=[jax.ShapeDtypeStruct((NP_, Q), jnp.float32),
              jax.ShapeDtypeStruct((NP_, Q), jnp.float32)],
    scratch_types=[
        pltpu.VMEM((RPT_ALL, 128), jnp.int32),
        pltpu.VMEM((128, Q), jnp.float32),
        pltpu.VMEM((ZR, Q), jnp.float32),
        pltpu.VMEM_SHARED((NP_, Q), jnp.float32),
        pltpu.SemaphoreType.DMA,
    ],
    compiler_params=pltpu.CompilerParams(use_tc_tiling_on_sc=False),
)
def _deg(dst2d, ones2d, out0, out1, dstbuf, onesbuf, zbuf, acc, sem):
    cc = lax.axis_index("c")
    ss = lax.axis_index("s")
    wid = ss * 2 + cc

    pltpu.sync_copy(dst2d.at[pl.ds(wid * RPT_ALL, RPT_ALL)], dstbuf)
    pltpu.sync_copy(ones2d, onesbuf)

    def zrow(i, _):
        zbuf[i, :] = _zeros16()
        return 0
    lax.fori_loop(0, ZR, zrow, 0)

    def zacc(q, _):
        pltpu.sync_copy(zbuf, acc.at[pl.ds(ss * RPN + q * ZR, ZR)])
        return 0
    lax.fori_loop(0, RPN // ZR, zacc, 0)
    plsc.subcore_barrier()

    def body(g, _):
        for b in range(4):
            pltpu.async_copy(onesbuf, acc.at[dstbuf.at[g * 4 + b]], sem,
                             add=True)
        @pl.when(g >= 2)
        def _():
            for b in range(4):
                pltpu.make_async_copy(onesbuf, acc.at[dstbuf.at[0]],
                                      sem).wait()
        return 0
    lax.fori_loop(0, RPT_ALL // 4, body, 0)
    for b in range(8):
        pltpu.make_async_copy(onesbuf, acc.at[dstbuf.at[0]], sem).wait()
    plsc.subcore_barrier()

    @pl.when(cc == 0)
    def _():
        pltpu.sync_copy(acc.at[pl.ds(ss * RPN, RPN)],
                        out0.at[pl.ds(ss * RPN, RPN)])

    @pl.when(cc == 1)
    def _():
        pltpu.sync_copy(acc.at[pl.ds(ss * RPN, RPN)],
                        out1.at[pl.ds(ss * RPN, RPN)])


@functools.partial(
    pl.kernel,
    mesh=_mesh,
    out_type=jax.ShapeDtypeStruct((2 * NP_, Q), jnp.float32),
    scratch_types=[
        pltpu.VMEM((CH, 128), jnp.int32),
        pltpu.VMEM((CH, 128), jnp.int32),
        pltpu.VMEM((128, Q), jnp.float32),
        pltpu.VMEM((128, Q), jnp.float32),
        pltpu.VMEM((128, Q), jnp.float32),
        pltpu.VMEM((128, Q), jnp.float32),
        pltpu.VMEM((ZR, Q), jnp.float32),
        pltpu.VMEM_SHARED((NP_, Q), jnp.float32),
        pltpu.SemaphoreType.DMA,
        pltpu.SemaphoreType.DMA,
        pltpu.SemaphoreType.DMA,
        pltpu.SemaphoreType.DMA,
        pltpu.SemaphoreType.DMA,
        pltpu.SemaphoreType.DMA,
        pltpu.SemaphoreType.DMA,
        pltpu.SemaphoreType.DMA,
    ],
    compiler_params=pltpu.CompilerParams(use_tc_tiling_on_sc=False),
)
def _prop(h4, srclo, srchi, dst2d, pout,
          srcbuf, dstbuf, r0, r1, r2, r3, zbuf, acc,
          g0, g1, g2, g3, s0, s1, s2, s3):
    cc = lax.axis_index("c")
    ss = lax.axis_index("s")
    rbufs = (r0, r1, r2, r3)
    gsems = (g0, g1, g2, g3)
    ssems = (s0, s1, s2, s3)

    def zrow(i, _):
        zbuf[i, :] = _zeros16()
        return 0
    lax.fori_loop(0, ZR, zrow, 0)

    def zacc(q, _):
        pltpu.sync_copy(zbuf, acc.at[pl.ds(ss * RPN + q * ZR, ZR)])
        return 0
    lax.fori_loop(0, RPN // ZR, zacc, 0)
    plsc.subcore_barrier()

    def chunk(qq, _):
        base = ss * RPT_SC + qq * CH

        @pl.when(cc == 0)
        def _():
            pltpu.sync_copy(srclo.at[pl.ds(base, CH)], srcbuf)

        @pl.when(cc == 1)
        def _():
            pltpu.sync_copy(srchi.at[pl.ds(base, CH)], srcbuf)

        pltpu.sync_copy(dst2d.at[pl.ds(base, CH)], dstbuf)

        for b in range(NBUF):
            pltpu.async_copy(h4.at[srcbuf.at[b]], rbufs[b], gsems[b])

        def body(g, _):
            for b in range(NBUF):
                j = g * NBUF + b
                rb = rbufs[b]
                pltpu.make_async_copy(h4.at[srcbuf.at[j]], rb,
                                      gsems[b]).wait()
                pltpu.async_copy(rb, acc.at[dstbuf.at[j]], ssems[b],
                                 add=True)
                pltpu.make_async_copy(rb, acc.at[dstbuf.at[j]],
                                      ssems[b]).wait()

                @pl.when(j + NBUF < CH)
                def _():
                    pltpu.async_copy(h4.at[srcbuf.at[j + NBUF]], rb,
                                     gsems[b])
            return 0
        lax.fori_loop(0, CH // NBUF, body, 0)
        return 0
    lax.fori_loop(0, NCH, chunk, 0)
    plsc.subcore_barrier()

    pltpu.sync_copy(acc.at[pl.ds(ss * RPN, RPN)],
                    pout.at[pl.ds(cc * NP_ + ss * RPN, RPN)])


def _embed_body(nt, xc, xg, xp, xr, xv, d0, d1, W1, b1, W2, b2, W0,
                h0, h1, h2, h3, dinv):
    oh = (nt[...] == lax.broadcasted_iota(jnp.int32, (RB, T), 1))
    e1 = jnp.dot(oh.astype(jnp.float32), W1[...],
                 preferred_element_type=jnp.float32) + b1[...]
    w2 = W2[...]
    e2 = (xc[...] * w2[0:1, :] + xg[...] * w2[1:2, :] + xp[...] * w2[2:3, :]
          + xr[...] * w2[3:4, :] + xv[...] * w2[4:5, :]) + b2[...]
    z = jnp.concatenate([e1, e2], axis=1)
    deg = 1.0 + d0[...][:, 0:1] + d1[...][:, 0:1]
    di = lax.rsqrt(deg)
    h = jnp.dot(z, W0[...], preferred_element_type=jnp.float32) * di
    h0[...] = h[:, 0 * Q:1 * Q]
    h1[...] = h[:, 1 * Q:2 * Q]
    h2[...] = h[:, 2 * Q:3 * Q]
    h3[...] = h[:, 3 * Q:4 * Q]
    dinv[...] = di


def _embed_call(nt2, cols, d0, d1, W1, b1r, W2, b2r, W0):
    col = pl.BlockSpec((RB, 1), lambda i: (i, 0))
    dq = pl.BlockSpec((RB, Q), lambda i: (i, 0))
    full = lambda s: pl.BlockSpec(s, lambda i: (0, 0))
    qs = jax.ShapeDtypeStruct((NP_, Q), jnp.float32)
    return pl.pallas_call(
        _embed_body,
        grid=(GRID,),
        in_specs=[col, col, col, col, col, col, dq, dq,
                  full((T, H)), full((1, H)), full((5, H)), full((1, H)),
                  full((2 * H, H))],
        out_specs=[dq, dq, dq, dq, col],
        out_shape=[qs, qs, qs, qs,
                   jax.ShapeDtypeStruct((NP_, 1), jnp.float32)],
    )(nt2, *cols, d0, d1, W1, b1r, W2, b2r, W0)


def _mid_body(p0, p1, p2, p3, h0, h1, h2, h3, dinv, bb, Wn,
              o0, o1, o2, o3):
    di = dinv[...]
    z = jnp.concatenate(
        [p0[...] + h0[...], p1[...] + h1[...],
         p2[...] + h2[...], p3[...] + h3[...]], axis=1)
    z = di * z + bb[...]
    z = jnp.maximum(z, 0.0)
    h = jnp.dot(z, Wn[...], preferred_element_type=jnp.float32) * di
    o0[...] = h[:, 0 * Q:1 * Q]
    o1[...] = h[:, 1 * Q:2 * Q]
    o2[...] = h[:, 2 * Q:3 * Q]
    o3[...] = h[:, 3 * Q:4 * Q]


def _mid_call(poutA, poutB, hq, dinv2, bb, Wn):
    dq = pl.BlockSpec((RB, Q), lambda i: (i, 0))
    pLO = pl.BlockSpec((RB, Q), lambda i: (i, 0))
    pHI = pl.BlockSpec((RB, Q), lambda i: (GRID + i, 0))
    col = pl.BlockSpec((RB, 1), lambda i: (i, 0))
    full = lambda s: pl.BlockSpec(s, lambda i: (0, 0))
    qs = jax.ShapeDtypeStruct((NP_, Q), jnp.float32)
    return pl.pallas_call(
        _mid_body,
        grid=(GRID,),
        in_specs=[pLO, pHI, pLO, pHI, dq, dq, dq, dq, col,
                  full((1, H)), full((H, H))],
        out_specs=[dq, dq, dq, dq],
        out_shape=[qs, qs, qs, qs],
    )(poutA, poutA, poutB, poutB, *hq, dinv2, bb, Wn)


def _fin_body(p0, p1, p2, p3, h0, h1, h2, h3, dinv, bb, bt, Wo, bo,
              accum, pred):
    i = pl.program_id(0)
    di = dinv[...]
    z = jnp.concatenate(
        [p0[...] + h0[...], p1[...] + h1[...],
         p2[...] + h2[...], p3[...] + h3[...]], axis=1)
    z = di * z + bb[...]
    m = (bt[...] == lax.broadcasted_iota(jnp.int32, (RB, NG), 1))
    zaug = jnp.concatenate(
        [z, jnp.ones((RB, 1), jnp.float32), jnp.zeros((RB, 63), jnp.float32)],
        axis=1)
    contrib = lax.dot_general(m.astype(jnp.float32), zaug,
                              (((0,), (0,)), ((), ())),
                              preferred_element_type=jnp.float32)

    @pl.when(i == 0)
    def _():
        accum[...] = jnp.zeros((NG, 128), jnp.float32)

    accum[...] += contrib

    @pl.when(i == GRID - 1)
    def _():
        a = accum[...]
        pooled = a[:, :H] / jnp.maximum(a[:, H:H + 1], 1.0)
        pred[...] = jnp.dot(pooled, Wo[...],
                            preferred_element_type=jnp.float32) + bo[...]


def _fin_call(poutA, poutB, hq, dinv2, bb, bt2, Wop, bop):
    dq = pl.BlockSpec((RB, Q), lambda i: (i, 0))
    pLO = pl.BlockSpec((RB, Q), lambda i: (i, 0))
    pHI = pl.BlockSpec((RB, Q), lambda i: (GRID + i, 0))
    col = pl.BlockSpec((RB, 1), lambda i: (i, 0))
    full = lambda s: pl.BlockSpec(s, lambda i: (0, 0))
    acc_spec = pl.BlockSpec((NG, 128), lambda i: (0, 0))
    _, pred = pl.pallas_call(
        _fin_body,
        grid=(GRID,),
        in_specs=[pLO, pHI, pLO, pHI, dq, dq, dq, dq, col,
                  full((1, H)), col, full((H, 128)), full((1, 128))],
        out_specs=[acc_spec, acc_spec],
        out_shape=[jax.ShapeDtypeStruct((NG, 128), jnp.float32),
                   jax.ShapeDtypeStruct((NG, 128), jnp.float32)],
    )(poutA, poutA, poutB, poutB, *hq, dinv2, bb, bt2, Wop, bop)
    return pred


def kernel(node_type, c, gm, pos, r, vid, edge_index, batch,
           W1, b1, W2, b2, gcn_params, Wout, bout):
    f32, i32 = jnp.float32, jnp.int32

    src = edge_index[0].astype(i32)
    dst = edge_index[1].astype(i32)
    src2d = jnp.concatenate([src, jnp.zeros((EP - E,), i32)]).reshape(EPR, 128)
    dst2d = jnp.concatenate([dst, jnp.full((EP - E,), N, i32)]).reshape(EPR, 128)
    srcq = [src2d + k * NP_ for k in range(4)]
    ones2d = jnp.ones((128, Q), f32)

    deg0, deg1 = _deg(dst2d, ones2d)

    padc = lambda a: jnp.pad(a.astype(f32), (0, NP_ - N)).reshape(NP_, 1)
    nt2 = jnp.pad(node_type.astype(i32), (0, NP_ - N)).reshape(NP_, 1)
    cols = [padc(c), padc(gm), padc(pos), padc(r), padc(vid)]
    bt2 = jnp.pad(batch.astype(i32), (0, NP_ - N),
                  constant_values=1 << 20).reshape(NP_, 1)

    W0 = gcn_params[0][0]
    *hq, dinv2 = _embed_call(nt2, cols, deg0, deg1,
                             W1, b1.reshape(1, H), W2, b2.reshape(1, H), W0)

    for l in range(NL):
        h4 = jnp.concatenate(hq, axis=0)
        poutA = _prop(h4, srcq[0], srcq[1], dst2d)
        poutB = _prop(h4, srcq[2], srcq[3], dst2d)
        bb = gcn_params[l][1].reshape(1, H)
        if l < NL - 1:
            hq = _mid_call(poutA, poutB, hq, dinv2, bb, gcn_params[l + 1][0])
        else:
            Wop = jnp.pad(Wout.astype(f32), ((0, 0), (0, 128 - 4)))
            bop = jnp.pad(bout.astype(f32), (0, 128 - 4)).reshape(1, 128)
            pred = _fin_call(poutA, poutB, hq, dinv2, bb, bt2, Wop, bop)

    return pred[:, :4]
